# baseline (speedup 1.0000x reference)
import jax
import jax.numpy as jnp
from jax import lax
from jax.experimental import pallas as pl
from jax.experimental.pallas import tpu as pltpu
from jax.experimental.pallas import tpu_sc as plsc

B = 16384
M = 50
D = 32
NW = 32
RPW = B // NW
CB = 32
NCH = RPW // CB
ROWS = CB * M
NFULL = ROWS // 128
TAIL = ROWS - NFULL * 128


def _tr_body(tin, tout):
    parts = [jnp.transpose(tin[:, 128 * a:128 * (a + 1)], (1, 0))
             for a in range(4)]
    tout[...] = jnp.reshape(jnp.concatenate(parts, axis=1), (16384,))


def _transpose_flat(tbl_t, nblk):
    return pl.pallas_call(
        _tr_body,
        grid=(nblk,),
        in_specs=[pl.BlockSpec((32, 512), lambda i: (0, i))],
        out_specs=pl.BlockSpec((16384,), lambda i: (i,)),
        out_shape=jax.ShapeDtypeStruct((nblk * 16384,), jnp.float32),
    )(tbl_t)


def _perm_rows(i):
    return ((i >> 9) << 9) | ((i & 127) << 2) | ((i >> 7) & 3)


def _bcast_lane(vec, lane):
    idx = jnp.full((16, 1), lane, jnp.int32)
    dn = lax.GatherDimensionNumbers(
        offset_dims=(), collapsed_slice_dims=(0,), start_index_map=(0,))
    return lax.gather(vec, idx, dn, (1,),
                      mode=lax.GatherScatterMode.PROMISE_IN_BOUNDS)


def _body(user_hbm, mem_hbm, utab_hbm, itab_hbm, cw_hbm, out_hbm,
          idx0, idx1, x0, x1, u0, u1, o0, o1, uidx, cw,
          insem0, insem1, outsem0, outsem1):
    idxb = (idx0, idx1)
    xb = (x0, x1)
    ub = (u0, u1)
    ob = (o0, o1)
    insem = (insem0, insem1)
    outsem = (outsem0, outsem1)

    wid = lax.axis_index("s") * 2 + lax.axis_index("c")
    row0 = wid * RPW

    pltpu.sync_copy(user_hbm.at[pl.ds(row0, RPW)], uidx)
    pltpu.sync_copy(cw_hbm, cw)

    def in_copies(g, s):
        cps = []
        for p in range(NFULL):
            cps.append(pltpu.make_async_copy(
                itab_hbm.at[idxb[s].at[pl.ds(p * 128, 128)]],
                xb[s].at[pl.ds(p * 128, 128)], insem[s]))
        cps.append(pltpu.make_async_copy(
            itab_hbm.at[idxb[s].at[pl.ds(NFULL * 128, TAIL)]],
            xb[s].at[pl.ds(NFULL * 128, TAIL)], insem[s]))
        cps.append(pltpu.make_async_copy(
            utab_hbm.at[uidx.at[pl.ds(g * CB, CB)]], ub[s], insem[s]))
        return cps

    def fire(g, s):
        ebase = row0 * M + g * ROWS
        pltpu.sync_copy(mem_hbm.at[pl.ds(ebase, ROWS)], idxb[s])
        for cp in in_copies(g, s):
            cp.start()

    def drain(g, s):
        for cp in in_copies(g, s):
            cp.wait()

    def out_copy(g, s):
        return pltpu.make_async_copy(
            ob[s], out_hbm.at[pl.ds(row0 + g * CB, CB)], outsem[s])

    def compute(g, s):
        xr = xb[s]
        ur = ub[s]
        orf = ob[s]
        cwv = [cw[pl.ds(16 * k, 16)] for k in range(4)]
        cb_vec = _bcast_lane(cwv[M // 16], M % 16)

        def brow(b, carry):
            rb = b * M
            acc0 = [jnp.zeros((16,), jnp.float32) for _ in range(4)]
            acc1 = [jnp.zeros((16,), jnp.float32) for _ in range(4)]
            for m in range(M):
                wv = _bcast_lane(cwv[m // 16], m % 16)
                r0 = xr[rb + m, pl.ds(0, 16)]
                r1 = xr[rb + m, pl.ds(16, 16)]
                acc0[m % 4] = acc0[m % 4] + wv * r0
                acc1[m % 4] = acc1[m % 4] + wv * r1
            drr0 = (acc0[0] + acc0[1]) + (acc0[2] + acc0[3]) + cb_vec
            drr1 = (acc1[0] + acc1[1]) + (acc1[2] + acc1[3]) + cb_vec
            ue0 = ur[b, pl.ds(0, 16)]
            ue1 = ur[b, pl.ds(16, 16)]
            orf[b, pl.ds(0, 16)] = ue0
            orf[b, pl.ds(16, 16)] = ue1
            orf[b, pl.ds(32, 16)] = ue0 * drr0
            orf[b, pl.ds(48, 16)] = ue1 * drr1
            orf[b, pl.ds(64, 16)] = drr0
            orf[b, pl.ds(80, 16)] = drr1
            return carry

        lax.fori_loop(0, CB, brow, 0)

    fire(0, 0)
    fire(1, 1)

    def outer(i, carry):
        for s in (0, 1):
            g = 2 * i + s

            @pl.when(g >= 2)
            def _():
                out_copy(g - 2, s).wait()

            drain(g, s)
            compute(g, s)
            out_copy(g, s).start()

            @pl.when(g + 2 < NCH)
            def _():
                fire(g + 2, s)
        return carry

    lax.fori_loop(0, NCH // 2, outer, 0)

    out_copy(NCH - 2, 0).wait()
    out_copy(NCH - 1, 1).wait()


def kernel(user, memory, user_table, item_table, conv_w, conv_b):
    mem_flat = _perm_rows(memory.astype(jnp.int32)).reshape(-1)
    user_i = _perm_rows(user.astype(jnp.int32))
    tab_i = _transpose_flat(item_table.T, 1954).reshape(-1, 32)
    tab_u = _transpose_flat(user_table.T, 196).reshape(-1, 32)
    cw_pad = jnp.concatenate(
        [conv_w.astype(jnp.float32),
         jnp.reshape(conv_b, (1,)).astype(jnp.float32),
         jnp.zeros((13,), jnp.float32)])

    mesh = plsc.VectorSubcoreMesh(core_axis_name="c", subcore_axis_name="s")
    f = pl.kernel(
        _body,
        out_type=jax.ShapeDtypeStruct((B, 3 * D), jnp.float32),
        mesh=mesh,
        compiler_params=pltpu.CompilerParams(use_tc_tiling_on_sc=False),
        scratch_types=[
            pltpu.VMEM((ROWS,), jnp.int32),
            pltpu.VMEM((ROWS,), jnp.int32),
            pltpu.VMEM((ROWS, D), jnp.float32),
            pltpu.VMEM((ROWS, D), jnp.float32),
            pltpu.VMEM((CB, D), jnp.float32),
            pltpu.VMEM((CB, D), jnp.float32),
            pltpu.VMEM((CB, 3 * D), jnp.float32),
            pltpu.VMEM((CB, 3 * D), jnp.float32),
            pltpu.VMEM((RPW,), jnp.int32),
            pltpu.VMEM((64,), jnp.float32),
            pltpu.SemaphoreType.DMA,
            pltpu.SemaphoreType.DMA,
            pltpu.SemaphoreType.DMA,
            pltpu.SemaphoreType.DMA,
        ],
    )
    return f(user_i, mem_flat, tab_u, tab_i, cw_pad)


# speedup vs baseline: 3.5203x; 3.5203x over previous
import jax
import jax.numpy as jnp
from jax import lax
from jax.experimental import pallas as pl
from jax.experimental.pallas import tpu as pltpu
from jax.experimental.pallas import tpu_sc as plsc

B = 16384
M = 50
D = 32
NW = 32
RPW = B // NW
CB = 32
NCH = RPW // CB
ROWS = CB * M
NFULL = ROWS // 128
TAIL = ROWS - NFULL * 128


def _tr_body(tin, tout):
    for j in range(32):
        parts = [jnp.transpose(
            tin[:, 512 * j + 128 * a:512 * j + 128 * (a + 1)], (1, 0))
            for a in range(4)]
        tout[pl.ds(16384 * j, 16384)] = jnp.reshape(
            jnp.concatenate(parts, axis=1), (16384,))


def _transpose_flat(tbl_t, nblk):
    return pl.pallas_call(
        _tr_body,
        grid=(nblk,),
        in_specs=[pl.BlockSpec((32, 16384), lambda i: (0, i))],
        out_specs=pl.BlockSpec((524288,), lambda i: (i,)),
        out_shape=jax.ShapeDtypeStruct((nblk * 524288,), jnp.float32),
    )(tbl_t)


def _perm_rows(i):
    return ((i >> 9) << 9) | ((i & 127) << 2) | ((i >> 7) & 3)


def _bcast_lane(vec, lane):
    idx = jnp.full((16, 1), lane, jnp.int32)
    dn = lax.GatherDimensionNumbers(
        offset_dims=(), collapsed_slice_dims=(0,), start_index_map=(0,))
    return lax.gather(vec, idx, dn, (1,),
                      mode=lax.GatherScatterMode.PROMISE_IN_BOUNDS)


def _body(user_hbm, mem_hbm, utab_hbm, itab_hbm, cw_hbm, out_hbm,
          idx0, idx1, x0, x1, u0, u1, o0, o1, uidx, cw,
          insem0, insem1, outsem0, outsem1):
    idxb = (idx0, idx1)
    xb = (x0, x1)
    ub = (u0, u1)
    ob = (o0, o1)
    insem = (insem0, insem1)
    outsem = (outsem0, outsem1)

    wid = lax.axis_index("s") * 2 + lax.axis_index("c")
    row0 = wid * RPW

    pltpu.sync_copy(user_hbm.at[pl.ds(row0, RPW)], uidx)
    pltpu.sync_copy(cw_hbm, cw)

    def in_copies(g, s):
        cps = []
        for p in range(NFULL):
            cps.append(pltpu.make_async_copy(
                itab_hbm.at[idxb[s].at[pl.ds(p * 128, 128)]],
                xb[s].at[pl.ds(p * 128, 128)], insem[s]))
        cps.append(pltpu.make_async_copy(
            itab_hbm.at[idxb[s].at[pl.ds(NFULL * 128, TAIL)]],
            xb[s].at[pl.ds(NFULL * 128, TAIL)], insem[s]))
        cps.append(pltpu.make_async_copy(
            utab_hbm.at[uidx.at[pl.ds(g * CB, CB)]], ub[s], insem[s]))
        return cps

    def fire(g, s):
        ebase = row0 * M + g * ROWS
        pltpu.sync_copy(mem_hbm.at[pl.ds(ebase, ROWS)], idxb[s])
        for cp in in_copies(g, s):
            cp.start()

    def drain(g, s):
        for cp in in_copies(g, s):
            cp.wait()

    def out_copy(g, s):
        return pltpu.make_async_copy(
            ob[s], out_hbm.at[pl.ds(row0 + g * CB, CB)], outsem[s])

    def compute(g, s):
        xr = xb[s]
        ur = ub[s]
        orf = ob[s]
        cwv = [cw[pl.ds(16 * k, 16)] for k in range(4)]
        cb_vec = _bcast_lane(cwv[M // 16], M % 16)

        def brow(b, carry):
            rb = b * M
            acc0 = [jnp.zeros((16,), jnp.float32) for _ in range(4)]
            acc1 = [jnp.zeros((16,), jnp.float32) for _ in range(4)]
            for m in range(M):
                wv = _bcast_lane(cwv[m // 16], m % 16)
                r0 = xr[rb + m, pl.ds(0, 16)]
                r1 = xr[rb + m, pl.ds(16, 16)]
                acc0[m % 4] = acc0[m % 4] + wv * r0
                acc1[m % 4] = acc1[m % 4] + wv * r1
            drr0 = (acc0[0] + acc0[1]) + (acc0[2] + acc0[3]) + cb_vec
            drr1 = (acc1[0] + acc1[1]) + (acc1[2] + acc1[3]) + cb_vec
            ue0 = ur[b, pl.ds(0, 16)]
            ue1 = ur[b, pl.ds(16, 16)]
            orf[b, pl.ds(0, 16)] = ue0
            orf[b, pl.ds(16, 16)] = ue1
            orf[b, pl.ds(32, 16)] = ue0 * drr0
            orf[b, pl.ds(48, 16)] = ue1 * drr1
            orf[b, pl.ds(64, 16)] = drr0
            orf[b, pl.ds(80, 16)] = drr1
            return carry

        lax.fori_loop(0, CB, brow, 0)

    fire(0, 0)
    fire(1, 1)

    def outer(i, carry):
        for s in (0, 1):
            g = 2 * i + s

            @pl.when(g >= 2)
            def _():
                out_copy(g - 2, s).wait()

            drain(g, s)
            compute(g, s)
            out_copy(g, s).start()

            @pl.when(g + 2 < NCH)
            def _():
                fire(g + 2, s)
        return carry

    lax.fori_loop(0, NCH // 2, outer, 0)

    out_copy(NCH - 2, 0).wait()
    out_copy(NCH - 1, 1).wait()


def kernel(user, memory, user_table, item_table, conv_w, conv_b):
    mem_flat = _perm_rows(memory.astype(jnp.int32)).reshape(-1)
    user_i = _perm_rows(user.astype(jnp.int32))
    tab_i = _transpose_flat(item_table.T, 62).reshape(-1, 32)
    tab_u = _transpose_flat(user_table.T, 7).reshape(-1, 32)
    cw_pad = jnp.concatenate(
        [conv_w.astype(jnp.float32),
         jnp.reshape(conv_b, (1,)).astype(jnp.float32),
         jnp.zeros((13,), jnp.float32)])

    mesh = plsc.VectorSubcoreMesh(core_axis_name="c", subcore_axis_name="s")
    f = pl.kernel(
        _body,
        out_type=jax.ShapeDtypeStruct((B, 3 * D), jnp.float32),
        mesh=mesh,
        compiler_params=pltpu.CompilerParams(use_tc_tiling_on_sc=False),
        scratch_types=[
            pltpu.VMEM((ROWS,), jnp.int32),
            pltpu.VMEM((ROWS,), jnp.int32),
            pltpu.VMEM((ROWS, D), jnp.float32),
            pltpu.VMEM((ROWS, D), jnp.float32),
            pltpu.VMEM((CB, D), jnp.float32),
            pltpu.VMEM((CB, D), jnp.float32),
            pltpu.VMEM((CB, 3 * D), jnp.float32),
            pltpu.VMEM((CB, 3 * D), jnp.float32),
            pltpu.VMEM((RPW,), jnp.int32),
            pltpu.VMEM((64,), jnp.float32),
            pltpu.SemaphoreType.DMA,
            pltpu.SemaphoreType.DMA,
            pltpu.SemaphoreType.DMA,
            pltpu.SemaphoreType.DMA,
        ],
    )
    return f(user_i, mem_flat, tab_u, tab_i, cw_pad)
